# Initial kernel scaffold; baseline (speedup 1.0000x reference)
import jax
import jax.numpy as jnp
from jax.experimental import pallas as pl
from jax.experimental.pallas import tpu as pltpu

_CUT = 5.0
_WID = 0.5
_P = 160000
_N = 10000
_E = 2000
_NB = _P // _E


def _body(i_s, j_s, spec_s, R_b, W_v, out_ref, an_ref, wrep_ref):
    b = pl.program_id(0)

    @pl.when(b == 0)
    def _init():
        out_ref[...] = jnp.zeros_like(out_ref)
        wrep_ref[...] = jnp.broadcast_to(W_v[...][:, :, None], wrep_ref.shape)

    R = R_b[...]
    x = R[:, 0:1]
    y = R[:, 1:2]
    z = R[:, 2:3]
    r2 = x * x + y * y + z * z + 1e-20
    r = jnp.sqrt(r2)
    inv = 1.0 / (r + 1e-10)
    ux = x * inv
    uy = y * inv
    uz = z * inv

    inner = _CUT - _WID
    t = jnp.clip((r - inner) / _WID, 0.0, 1.0)
    smooth = 0.5 * (1.0 + jnp.cos(jnp.pi * t))
    cut = jnp.where(r < inner, 1.0, jnp.where(r < _CUT, smooth, 0.0))

    nvec = jnp.arange(1, 10, dtype=jnp.float32)[None, :] * (jnp.pi / _CUT)
    rad = jnp.sqrt(2.0 / _CUT) * jnp.sin(r * nvec) * inv * cut

    zz2 = uz * uz
    ang = jnp.concatenate([
        0.28209479177387814 * jnp.ones_like(x),
        0.4886025119029199 * uy,
        0.4886025119029199 * uz,
        0.4886025119029199 * ux,
        1.0925484305920792 * ux * uy,
        1.0925484305920792 * uy * uz,
        0.31539156525252005 * (3.0 * zz2 - 1.0),
        1.0925484305920792 * ux * uz,
        0.5462742152960396 * (ux * ux - uy * uy),
        0.5900435899266435 * uy * (3.0 * ux * ux - uy * uy),
        2.890611442640554 * ux * uy * uz,
        0.4570457994644658 * uy * (5.0 * zz2 - 1.0),
        0.3731763325901154 * uz * (5.0 * zz2 - 3.0),
        0.4570457994644658 * ux * (5.0 * zz2 - 1.0),
        1.445305721320277 * uz * (ux * ux - uy * uy),
        0.5900435899266435 * ux * (ux * ux - 3.0 * uy * uy),
    ], axis=1)

    for n in range(9):
        an_ref[:, 16 * n:16 * (n + 1)] = rad[:, n:n + 1] * ang

    def loop(p, carry):
        idx = i_s[0, 0, p]
        jp = j_s[0, 0, p]
        sp = spec_s[jp]
        wb = wrep_ref[sp]
        an = an_ref[p]
        out_ref[idx] = out_ref[idx] + wb * an[None, :]
        return carry

    jax.lax.fori_loop(0, _E, loop, 0)


def kernel(R_ij, i, j, species, W_species):
    i3 = i.astype(jnp.int32).reshape(_NB, 1, _E)
    j3 = j.astype(jnp.int32).reshape(_NB, 1, _E)
    acc = pl.pallas_call(
        _body,
        grid=(_NB,),
        in_specs=[
            pl.BlockSpec((1, 1, _E), lambda b: (b, 0, 0), memory_space=pltpu.SMEM),
            pl.BlockSpec((1, 1, _E), lambda b: (b, 0, 0), memory_space=pltpu.SMEM),
            pl.BlockSpec(memory_space=pltpu.SMEM),
            pl.BlockSpec((_E, 3), lambda b: (b, 0)),
            pl.BlockSpec((100, 4), lambda b: (0, 0)),
        ],
        out_specs=pl.BlockSpec((_N, 4, 144), lambda b: (0, 0, 0)),
        out_shape=jax.ShapeDtypeStruct((_N, 4, 144), jnp.float32),
        scratch_shapes=[
            pltpu.VMEM((_E, 144), jnp.float32),
            pltpu.VMEM((100, 4, 144), jnp.float32),
        ],
    )(i3, j3, species.astype(jnp.int32), R_ij, W_species)
    t = acc.reshape(_N, 4, 9, 16).transpose(0, 3, 2, 1)
    return (t[:, 0:1], t[:, 1:4], t[:, 4:9], t[:, 9:16])


# speedup vs baseline: 24.4775x; 24.4775x over previous
import jax
import jax.numpy as jnp
from jax.experimental import pallas as pl
from jax.experimental.pallas import tpu as pltpu

_CUT = 5.0
_WID = 0.5
_P = 160000
_N = 10000
_E = 2000
_NB = _P // _E


def _body(i_s, j_s, spec_s, R_b, W_v, out_ref, outt_ref, an_ref, ant_ref,
          wrep_ref, wt_ref):
    b = pl.program_id(0)

    @pl.when(b == 0)
    def _init():
        out_ref[...] = jnp.zeros_like(out_ref)
        outt_ref[...] = jnp.zeros_like(outt_ref)
        wrep_ref[...] = jnp.broadcast_to(W_v[...][:, :, None], wrep_ref.shape)
        for c in range(4):
            wt_ref[:, 16 * c:16 * (c + 1)] = jnp.broadcast_to(
                W_v[:, c:c + 1], (100, 16))

    R = R_b[...]
    x = R[:, 0:1]
    y = R[:, 1:2]
    z = R[:, 2:3]
    r2 = x * x + y * y + z * z + 1e-20
    r = jnp.sqrt(r2)
    inv = 1.0 / (r + 1e-10)
    ux = x * inv
    uy = y * inv
    uz = z * inv

    inner = _CUT - _WID
    t = jnp.clip((r - inner) / _WID, 0.0, 1.0)
    smooth = 0.5 * (1.0 + jnp.cos(jnp.pi * t))
    cut = jnp.where(r < inner, 1.0, jnp.where(r < _CUT, smooth, 0.0))

    nvec = (jax.lax.broadcasted_iota(jnp.int32, (1, 9), 1) + 1).astype(
        jnp.float32) * (jnp.pi / _CUT)
    rad = jnp.sqrt(2.0 / _CUT) * jnp.sin(r * nvec) * inv * cut

    zz2 = uz * uz
    ang = jnp.concatenate([
        0.28209479177387814 * jnp.ones_like(x),
        0.4886025119029199 * uy,
        0.4886025119029199 * uz,
        0.4886025119029199 * ux,
        1.0925484305920792 * ux * uy,
        1.0925484305920792 * uy * uz,
        0.31539156525252005 * (3.0 * zz2 - 1.0),
        1.0925484305920792 * ux * uz,
        0.5462742152960396 * (ux * ux - uy * uy),
        0.5900435899266435 * uy * (3.0 * ux * ux - uy * uy),
        2.890611442640554 * ux * uy * uz,
        0.4570457994644658 * uy * (5.0 * zz2 - 1.0),
        0.3731763325901154 * uz * (5.0 * zz2 - 3.0),
        0.4570457994644658 * ux * (5.0 * zz2 - 1.0),
        1.445305721320277 * uz * (ux * ux - uy * uy),
        0.5900435899266435 * ux * (ux * ux - 3.0 * uy * uy),
    ], axis=1)

    for n in range(8):
        an_ref[:, 16 * n:16 * (n + 1)] = rad[:, n:n + 1] * ang
    rt = rad[:, 8:9] * ang
    for c in range(4):
        ant_ref[:, 16 * c:16 * (c + 1)] = rt

    def loop(p, carry):
        idx = i_s[0, 0, p]
        jp = j_s[0, 0, p]
        sp = spec_s[jp]
        an = an_ref[p]
        out_ref[idx] = out_ref[idx] + wrep_ref[sp] * an[None, :]
        outt_ref[idx] = outt_ref[idx] + wt_ref[sp] * ant_ref[p]
        return carry

    jax.lax.fori_loop(0, _E, loop, 0)


def kernel(R_ij, i, j, species, W_species):
    i3 = i.astype(jnp.int32).reshape(_NB, 1, _E)
    j3 = j.astype(jnp.int32).reshape(_NB, 1, _E)
    acc, acct = pl.pallas_call(
        _body,
        grid=(_NB,),
        in_specs=[
            pl.BlockSpec((1, 1, _E), lambda b: (b, 0, 0), memory_space=pltpu.SMEM),
            pl.BlockSpec((1, 1, _E), lambda b: (b, 0, 0), memory_space=pltpu.SMEM),
            pl.BlockSpec(memory_space=pltpu.SMEM),
            pl.BlockSpec((_E, 3), lambda b: (b, 0)),
            pl.BlockSpec((100, 4), lambda b: (0, 0)),
        ],
        out_specs=[
            pl.BlockSpec((_N, 4, 128), lambda b: (0, 0, 0)),
            pl.BlockSpec((_N, 64), lambda b: (0, 0)),
        ],
        out_shape=[
            jax.ShapeDtypeStruct((_N, 4, 128), jnp.float32),
            jax.ShapeDtypeStruct((_N, 64), jnp.float32),
        ],
        scratch_shapes=[
            pltpu.VMEM((_E, 128), jnp.float32),
            pltpu.VMEM((_E, 64), jnp.float32),
            pltpu.VMEM((100, 4, 128), jnp.float32),
            pltpu.VMEM((100, 64), jnp.float32),
        ],
    )(i3, j3, species.astype(jnp.int32), R_ij, W_species)
    main = acc.reshape(_N, 4, 8, 16)
    tail = acct.reshape(_N, 4, 1, 16)
    t = jnp.concatenate([main, tail], axis=2).transpose(0, 3, 2, 1)
    return (t[:, 0:1], t[:, 1:4], t[:, 4:9], t[:, 9:16])


# speedup vs baseline: 27.1537x; 1.1093x over previous
import functools

import jax
import jax.numpy as jnp
from jax import lax
from jax.experimental import pallas as pl
from jax.experimental.pallas import tpu as pltpu
from jax.experimental.pallas import tpu_sc as plsc

_CUT = 5.0
_WID = 0.5
_P = 160000
_P2 = 163840
_N = 10000
_NACC = 10112
_E = 2048
_NB = _P2 // _E
_G = 128
_NBATCH = _P2 // _G
_BPT = _NBATCH // 16
_BPT_E = _NBATCH // 32


def _tc_body(R_b, spec_v, W_v, an_ref, ant_ref, nembT_ref):
    b = pl.program_id(0)

    @pl.when(b == 0)
    def _emb():
        iot = lax.broadcasted_iota(jnp.int32, (100, _N), 0)
        oh = (iot == spec_v[...]).astype(jnp.float32)
        nembT_ref[...] = lax.dot_general(
            W_v[...], oh, (((0,), (0,)), ((), ())),
            preferred_element_type=jnp.float32)

    R = R_b[...]
    x = R[:, 0:1]
    y = R[:, 1:2]
    z = R[:, 2:3]
    r2 = x * x + y * y + z * z + 1e-20
    r = jnp.sqrt(r2)
    inv = 1.0 / (r + 1e-10)
    ux = x * inv
    uy = y * inv
    uz = z * inv

    inner = _CUT - _WID
    t = jnp.clip((r - inner) / _WID, 0.0, 1.0)
    smooth = 0.5 * (1.0 + jnp.cos(jnp.pi * t))
    cut = jnp.where(r < inner, 1.0, jnp.where(r < _CUT, smooth, 0.0))

    nvec = (lax.broadcasted_iota(jnp.int32, (1, 9), 1) + 1).astype(
        jnp.float32) * (jnp.pi / _CUT)
    rad = jnp.sqrt(2.0 / _CUT) * jnp.sin(r * nvec) * inv * cut

    zz2 = uz * uz
    ang = jnp.concatenate([
        0.28209479177387814 * jnp.ones_like(x),
        0.4886025119029199 * uy,
        0.4886025119029199 * uz,
        0.4886025119029199 * ux,
        1.0925484305920792 * ux * uy,
        1.0925484305920792 * uy * uz,
        0.31539156525252005 * (3.0 * zz2 - 1.0),
        1.0925484305920792 * ux * uz,
        0.5462742152960396 * (ux * ux - uy * uy),
        0.5900435899266435 * uy * (3.0 * ux * ux - uy * uy),
        2.890611442640554 * ux * uy * uz,
        0.4570457994644658 * uy * (5.0 * zz2 - 1.0),
        0.3731763325901154 * uz * (5.0 * zz2 - 3.0),
        0.4570457994644658 * ux * (5.0 * zz2 - 1.0),
        1.445305721320277 * uz * (ux * ux - uy * uy),
        0.5900435899266435 * ux * (ux * ux - 3.0 * uy * uy),
    ], axis=1)

    for n in range(8):
        an_ref[:, 16 * n:16 * (n + 1)] = rad[:, n:n + 1] * ang
    ant_ref[...] = rad[:, 8:9] * ang


def _tc_stage(R_pad, species, W_species):
    return pl.pallas_call(
        _tc_body,
        grid=(_NB,),
        in_specs=[
            pl.BlockSpec((_E, 3), lambda b: (b, 0)),
            pl.BlockSpec((1, _N), lambda b: (0, 0)),
            pl.BlockSpec((100, 4), lambda b: (0, 0)),
        ],
        out_specs=[
            pl.BlockSpec((_E, 128), lambda b: (b, 0)),
            pl.BlockSpec((_E, 16), lambda b: (b, 0)),
            pl.BlockSpec((4, _N), lambda b: (0, 0)),
        ],
        out_shape=[
            jax.ShapeDtypeStruct((_P2, 128), jnp.float32),
            jax.ShapeDtypeStruct((_P2, 16), jnp.float32),
            jax.ShapeDtypeStruct((4, _N), jnp.float32),
        ],
    )(R_pad, species.reshape(1, _N).astype(jnp.int32), W_species)


def _sc_full(an_hbm, ant_hbm, i3_hbm, j2_hbm, nembw_hbm, nemba_hbm,
             outm_hbm, outt_hbm,
             anbuf, tbuf, ebuf, ibuf, jbuf, sem, acc_sh):
    cid = lax.axis_index("c")
    sid = lax.axis_index("s")
    base = sid * 632

    def zero_anbuf():
        def zrow(e, _):
            for n in range(8):
                anbuf[e, 16 * n:16 * (n + 1)] = jnp.zeros((16,), jnp.float32)
            return 0
        lax.fori_loop(0, _G, zrow, 0)

    def zero_acc():
        for q in range(4):
            pltpu.sync_copy(anbuf, acc_sh.at[pl.ds(base + q * _G, _G)])
        pltpu.sync_copy(anbuf.at[pl.ds(0, 120)],
                        acc_sh.at[pl.ds(base + 4 * _G, 120)])
        plsc.subcore_barrier()

    def scatter_batch():
        pltpu.sync_copy(anbuf, acc_sh.at[ibuf.at[0]], add=True)

    for chunk in range(2):
        c_val = cid * 2 + chunk
        zero_anbuf()
        zero_acc()

        def batch_body(b, _):
            gb = sid * _BPT + b
            pltpu.sync_copy(an_hbm.at[pl.ds(gb * _G, _G)], anbuf)
            pltpu.sync_copy(i3_hbm.at[gb], ibuf)
            pltpu.sync_copy(j2_hbm.at[gb], jbuf)
            pltpu.async_copy(nembw_hbm.at[c_val].at[jbuf], ebuf, sem).wait()

            def edge(p, _):
                s = ebuf[p, 0:16]
                for n in range(8):
                    anbuf[p, 16 * n:16 * (n + 1)] = (
                        anbuf[p, 16 * n:16 * (n + 1)] * s)
                return 0
            lax.fori_loop(0, _G, edge, 0)
            scatter_batch()
            return 0

        lax.fori_loop(0, _BPT, batch_body, 0)
        plsc.subcore_barrier()
        pltpu.sync_copy(acc_sh.at[pl.ds(base, 632)],
                        outm_hbm.at[c_val, pl.ds(base, 632)])
        plsc.subcore_barrier()

    zero_anbuf()
    zero_acc()

    def tail_body(b, _):
        gb = cid * (_NBATCH // 2) + sid * _BPT_E + b
        pltpu.sync_copy(ant_hbm.at[pl.ds(gb * _G, _G)], tbuf)
        pltpu.sync_copy(i3_hbm.at[gb], ibuf)
        pltpu.sync_copy(j2_hbm.at[gb], jbuf)
        pltpu.async_copy(nemba_hbm.at[jbuf], ebuf, sem).wait()

        def edge(p, _):
            tv = tbuf[p]
            for c in range(4):
                anbuf[p, 16 * c:16 * (c + 1)] = ebuf[p, 16 * c:16 * (c + 1)] * tv
            return 0
        lax.fori_loop(0, _G, edge, 0)
        scatter_batch()
        return 0

    lax.fori_loop(0, _BPT_E, tail_body, 0)
    plsc.subcore_barrier()
    pltpu.sync_copy(acc_sh.at[pl.ds(base, 632)],
                    outt_hbm.at[cid, pl.ds(base, 632)])


def _sc_stage(an, ant, i3, j2, nembw, nemba):
    mesh = plsc.VectorSubcoreMesh(core_axis_name="c", subcore_axis_name="s")
    run = functools.partial(
        pl.kernel,
        mesh=mesh,
        out_type=[
            jax.ShapeDtypeStruct((4, _NACC, 128), jnp.float32),
            jax.ShapeDtypeStruct((2, _NACC, 128), jnp.float32),
        ],
        scratch_types=[
            pltpu.VMEM((_G, 128), jnp.float32),
            pltpu.VMEM((_G, 16), jnp.float32),
            pltpu.VMEM((_G, 128), jnp.float32),
            pltpu.VMEM((1, 128), jnp.int32),
            pltpu.VMEM((_G,), jnp.int32),
            pltpu.SemaphoreType.DMA,
            pltpu.VMEM_SHARED((_NACC, 128), jnp.float32),
        ],
    )
    return run(_sc_full)(an, ant, i3, j2, nembw, nemba)


def kernel(R_ij, i, j, species, W_species):
    pad = _P2 - _P
    R_far = jnp.concatenate(
        [jnp.full((pad, 1), 10.0, R_ij.dtype), jnp.zeros((pad, 2), R_ij.dtype)],
        axis=1)
    R_pad = jnp.concatenate([R_ij, R_far], axis=0)
    i_pad = jnp.concatenate(
        [i.astype(jnp.int32), jnp.full((pad,), _N, jnp.int32)])
    j_pad = jnp.concatenate(
        [j.astype(jnp.int32), jnp.zeros((pad,), jnp.int32)])
    i3 = i_pad.reshape(_NBATCH, 1, 128)
    j2 = j_pad.reshape(_NBATCH, _G)

    an, ant, nembT = _tc_stage(R_pad, species, W_species)
    nembw = jnp.broadcast_to(nembT[:, :, None], (4, _N, 128))
    nemba = jnp.concatenate([
        jnp.broadcast_to(nembT.T[:, :, None], (_N, 4, 16)).reshape(_N, 64),
        jnp.zeros((_N, 64), jnp.float32),
    ], axis=1)

    outm, outt = _sc_stage(an, ant, i3, j2, nembw, nemba)

    tm = outm[:, :_N, :].reshape(4, _N, 8, 16).transpose(1, 3, 2, 0)
    tail = (outt[0, :_N, :64] + outt[1, :_N, :64]).reshape(_N, 4, 16)
    tt = tail.transpose(0, 2, 1)[:, :, None, :]
    t = jnp.concatenate([tm, tt], axis=2)
    return (t[:, 0:1], t[:, 1:4], t[:, 4:9], t[:, 9:16])


# speedup vs baseline: 31.2536x; 1.1510x over previous
import functools

import jax
import jax.numpy as jnp
from jax import lax
from jax.experimental import pallas as pl
from jax.experimental.pallas import tpu as pltpu
from jax.experimental.pallas import tpu_sc as plsc

_CUT = 5.0
_WID = 0.5
_P = 160000
_P2 = 163840
_N = 10000
_NACC = 10112
_E = 2048
_NB = _P2 // _E
_G = 64
_NBATCH = _P2 // _G
_BPT = _NBATCH // 16
_BPT_E = _NBATCH // 32


def _tc_body(R_b, spec_v, W_v, an_ref, ant_ref, nembT_ref):
    b = pl.program_id(0)

    @pl.when(b == 0)
    def _emb():
        iot = lax.broadcasted_iota(jnp.int32, (100, _N), 0)
        oh = (iot == spec_v[...]).astype(jnp.float32)
        nembT_ref[...] = lax.dot_general(
            W_v[...], oh, (((0,), (0,)), ((), ())),
            preferred_element_type=jnp.float32)

    R = R_b[...]
    x = R[:, 0:1]
    y = R[:, 1:2]
    z = R[:, 2:3]
    r2 = x * x + y * y + z * z + 1e-20
    r = jnp.sqrt(r2)
    inv = 1.0 / (r + 1e-10)
    ux = x * inv
    uy = y * inv
    uz = z * inv

    inner = _CUT - _WID
    t = jnp.clip((r - inner) / _WID, 0.0, 1.0)
    smooth = 0.5 * (1.0 + jnp.cos(jnp.pi * t))
    cut = jnp.where(r < inner, 1.0, jnp.where(r < _CUT, smooth, 0.0))

    nvec = (lax.broadcasted_iota(jnp.int32, (1, 9), 1) + 1).astype(
        jnp.float32) * (jnp.pi / _CUT)
    rad = jnp.sqrt(2.0 / _CUT) * jnp.sin(r * nvec) * inv * cut

    zz2 = uz * uz
    ang = jnp.concatenate([
        0.28209479177387814 * jnp.ones_like(x),
        0.4886025119029199 * uy,
        0.4886025119029199 * uz,
        0.4886025119029199 * ux,
        1.0925484305920792 * ux * uy,
        1.0925484305920792 * uy * uz,
        0.31539156525252005 * (3.0 * zz2 - 1.0),
        1.0925484305920792 * ux * uz,
        0.5462742152960396 * (ux * ux - uy * uy),
        0.5900435899266435 * uy * (3.0 * ux * ux - uy * uy),
        2.890611442640554 * ux * uy * uz,
        0.4570457994644658 * uy * (5.0 * zz2 - 1.0),
        0.3731763325901154 * uz * (5.0 * zz2 - 3.0),
        0.4570457994644658 * ux * (5.0 * zz2 - 1.0),
        1.445305721320277 * uz * (ux * ux - uy * uy),
        0.5900435899266435 * ux * (ux * ux - 3.0 * uy * uy),
    ], axis=1)

    for n in range(8):
        an_ref[:, 16 * n:16 * (n + 1)] = rad[:, n:n + 1] * ang
    ant_ref[...] = rad[:, 8:9] * ang


def _tc_stage(R_pad, species, W_species):
    return pl.pallas_call(
        _tc_body,
        grid=(_NB,),
        in_specs=[
            pl.BlockSpec((_E, 3), lambda b: (b, 0)),
            pl.BlockSpec((1, _N), lambda b: (0, 0)),
            pl.BlockSpec((100, 4), lambda b: (0, 0)),
        ],
        out_specs=[
            pl.BlockSpec((_E, 128), lambda b: (b, 0)),
            pl.BlockSpec((_E, 16), lambda b: (b, 0)),
            pl.BlockSpec((4, _N), lambda b: (0, 0)),
        ],
        out_shape=[
            jax.ShapeDtypeStruct((_P2, 128), jnp.float32),
            jax.ShapeDtypeStruct((_P2, 16), jnp.float32),
            jax.ShapeDtypeStruct((4, _N), jnp.float32),
        ],
    )(R_pad, species.reshape(1, _N).astype(jnp.int32), W_species)


def _sc_full(an_hbm, ant_hbm, i3_hbm, j2_hbm, nembw_hbm, nemba_hbm,
             outm_hbm, outt_hbm,
             an0, an1, eb0, eb1, ib0, ib1, jb0, jb1, tb0, tb1,
             ls0, ls1, gs0, gs1, ss0, ss1, acc_sh):
    cid = lax.axis_index("c")
    sid = lax.axis_index("s")
    base = sid * 632
    anb = (an0, an1)
    ebb = (eb0, eb1)
    ibb = (ib0, ib1)
    jbb = (jb0, jb1)
    tbb = (tb0, tb1)
    lss = (ls0, ls1)
    gss = (gs0, gs1)
    sss = (ss0, ss1)

    def zero_an0():
        def zrow(e, _):
            for n in range(8):
                an0[e, 16 * n:16 * (n + 1)] = jnp.zeros((16,), jnp.float32)
            return 0
        lax.fori_loop(0, _G, zrow, 0)

    def zero_acc():
        for q in range(9):
            pltpu.sync_copy(an0, acc_sh.at[pl.ds(base + q * _G, _G)])
        pltpu.sync_copy(an0.at[pl.ds(0, 56)],
                        acc_sh.at[pl.ds(base + 9 * _G, 56)])
        plsc.subcore_barrier()

    def lin_start(q, gb, tail):
        src = ant_hbm if tail else an_hbm
        dst = tbb[q] if tail else anb[q]
        pltpu.async_copy(src.at[pl.ds(gb * _G, _G)], dst, lss[q])
        pltpu.async_copy(i3_hbm.at[gb], ibb[q], lss[q])
        pltpu.async_copy(j2_hbm.at[gb], jbb[q], lss[q])

    def lin_wait(q, gb, tail):
        src = ant_hbm if tail else an_hbm
        dst = tbb[q] if tail else anb[q]
        pltpu.make_async_copy(src.at[pl.ds(gb * _G, _G)], dst, lss[q]).wait()
        pltpu.make_async_copy(i3_hbm.at[gb], ibb[q], lss[q]).wait()
        pltpu.make_async_copy(j2_hbm.at[gb], jbb[q], lss[q]).wait()

    def gather_start(q, c_val, tail):
        if tail:
            pltpu.async_copy(nemba_hbm.at[jbb[q]], ebb[q], gss[q])
        else:
            pltpu.async_copy(nembw_hbm.at[c_val].at[jbb[q]], ebb[q], gss[q])

    def gather_wait(q, c_val, tail):
        src = nemba_hbm.at[jbb[q]] if tail else nembw_hbm.at[c_val].at[jbb[q]]
        pltpu.make_async_copy(src, ebb[q], gss[q]).wait()

    def scatter_start(q):
        pltpu.async_copy(anb[q], acc_sh.at[ibb[q].at[0]], sss[q], add=True)

    def scatter_wait(q):
        pltpu.make_async_copy(anb[q], acc_sh.at[ibb[q].at[0]], sss[q]).wait()

    def compute_main(q):
        buf = anb[q]
        emb = ebb[q]

        def edge(p, _):
            sv = emb[p, 0:16]
            for n in range(8):
                buf[p, 16 * n:16 * (n + 1)] = buf[p, 16 * n:16 * (n + 1)] * sv
            return 0
        lax.fori_loop(0, _G, edge, 0)

    def compute_tail(q):
        buf = anb[q]
        emb = ebb[q]
        tb = tbb[q]

        def edge(p, _):
            tv = tb[p]
            for c in range(4):
                buf[p, 16 * c:16 * (c + 1)] = emb[p, 16 * c:16 * (c + 1)] * tv
            return 0
        lax.fori_loop(0, _G, edge, 0)

    def run_chunk(c_val, tail, gbase, nbatch):
        compute = compute_tail if tail else compute_main
        npair = nbatch // 2
        lin_start(0, gbase, tail)

        def pair(t, _):
            gb0 = gbase + 2 * t
            gb1 = gb0 + 1
            lin_wait(0, gb0, tail)
            gather_start(0, c_val, tail)

            @pl.when(t > 0)
            def _():
                scatter_wait(1)
            lin_start(1, gb1, tail)
            gather_wait(0, c_val, tail)
            compute(0)
            scatter_start(0)

            lin_wait(1, gb1, tail)
            gather_start(1, c_val, tail)
            scatter_wait(0)

            @pl.when(t + 1 < npair)
            def _():
                lin_start(0, gb0 + 2, tail)
            gather_wait(1, c_val, tail)
            compute(1)
            scatter_start(1)
            return 0

        lax.fori_loop(0, npair, pair, 0)
        scatter_wait(1)
        plsc.subcore_barrier()

    for chunk in range(2):
        c_val = cid * 2 + chunk
        zero_an0()
        zero_acc()
        run_chunk(c_val, False, sid * _BPT, _BPT)
        pltpu.sync_copy(acc_sh.at[pl.ds(base, 632)],
                        outm_hbm.at[c_val, pl.ds(base, 632)])
        plsc.subcore_barrier()

    zero_an0()
    zero_acc()
    run_chunk(0, True, cid * (_NBATCH // 2) + sid * _BPT_E, _BPT_E)
    pltpu.sync_copy(acc_sh.at[pl.ds(base, 632)],
                    outt_hbm.at[cid, pl.ds(base, 632)])


def _sc_stage(an, ant, i3, j2, nembw, nemba):
    mesh = plsc.VectorSubcoreMesh(core_axis_name="c", subcore_axis_name="s")
    run = functools.partial(
        pl.kernel,
        mesh=mesh,
        out_type=[
            jax.ShapeDtypeStruct((4, _NACC, 128), jnp.float32),
            jax.ShapeDtypeStruct((2, _NACC, 128), jnp.float32),
        ],
        scratch_types=[
            pltpu.VMEM((_G, 128), jnp.float32),
            pltpu.VMEM((_G, 128), jnp.float32),
            pltpu.VMEM((_G, 128), jnp.float32),
            pltpu.VMEM((_G, 128), jnp.float32),
            pltpu.VMEM((1, _G), jnp.int32),
            pltpu.VMEM((1, _G), jnp.int32),
            pltpu.VMEM((_G,), jnp.int32),
            pltpu.VMEM((_G,), jnp.int32),
            pltpu.VMEM((_G, 16), jnp.float32),
            pltpu.VMEM((_G, 16), jnp.float32),
            pltpu.SemaphoreType.DMA,
            pltpu.SemaphoreType.DMA,
            pltpu.SemaphoreType.DMA,
            pltpu.SemaphoreType.DMA,
            pltpu.SemaphoreType.DMA,
            pltpu.SemaphoreType.DMA,
            pltpu.VMEM_SHARED((_NACC, 128), jnp.float32),
        ],
    )
    return run(_sc_full)(an, ant, i3, j2, nembw, nemba)


def kernel(R_ij, i, j, species, W_species):
    pad = _P2 - _P
    R_far = jnp.concatenate(
        [jnp.full((pad, 1), 10.0, R_ij.dtype), jnp.zeros((pad, 2), R_ij.dtype)],
        axis=1)
    R_pad = jnp.concatenate([R_ij, R_far], axis=0)
    i_pad = jnp.concatenate(
        [i.astype(jnp.int32), jnp.full((pad,), _N, jnp.int32)])
    j_pad = jnp.concatenate(
        [j.astype(jnp.int32), jnp.zeros((pad,), jnp.int32)])
    i3 = i_pad.reshape(_NBATCH, 1, _G)
    j2 = j_pad.reshape(_NBATCH, _G)

    an, ant, nembT = _tc_stage(R_pad, species, W_species)
    nembw = jnp.broadcast_to(nembT[:, :, None], (4, _N, 128))
    nemba = jnp.concatenate([
        jnp.broadcast_to(nembT.T[:, :, None], (_N, 4, 16)).reshape(_N, 64),
        jnp.zeros((_N, 64), jnp.float32),
    ], axis=1)

    outm, outt = _sc_stage(an, ant, i3, j2, nembw, nemba)

    tm = outm[:, :_N, :].reshape(4, _N, 8, 16).transpose(1, 3, 2, 0)
    tail = (outt[0, :_N, :64] + outt[1, :_N, :64]).reshape(_N, 4, 16)
    tt = tail.transpose(0, 2, 1)[:, :, None, :]
    t = jnp.concatenate([tm, tt], axis=2)
    return (t[:, 0:1], t[:, 1:4], t[:, 4:9], t[:, 9:16])
